# Initial kernel scaffold; baseline (speedup 1.0000x reference)
import jax
import jax.numpy as jnp
from jax.experimental import pallas as pl


def kernel(node_feats, edge_index, edge_feats, W_l, b_l, W_r, b_r, W_e, att, bias):
    raise NotImplementedError("write your pallas kernel here")


# speedup vs baseline: 5.7230x; 5.7230x over previous
import functools

import jax
import jax.numpy as jnp
from jax import lax
from jax.experimental import pallas as pl
from jax.experimental.pallas import tpu as pltpu
from jax.experimental.pallas import tpu_sc as plsc

N = 10000
E = 320000
D = 128
C = 128
ED = 16

NUM_CORES = 2
NUM_SUBCORES = 16
NW = NUM_CORES * NUM_SUBCORES
EDGES_PER_W = E // NW
CHUNK = 40
NCHUNK = EDGES_PER_W // CHUNK
NPAD = 10240
ROWS_PER_TILE = NPAD // NUM_SUBCORES
ZROWS = 128
DN_ROWS = NPAD // 8
DN_PER_TILE = DN_ROWS // NUM_SUBCORES


def _lin_body(x_ref, wl_ref, bl_ref, wr_ref, br_ref, xl_ref, xr_ref):
    x = x_ref[...]
    xl_ref[...] = jnp.dot(x, wl_ref[...], preferred_element_type=jnp.float32) + bl_ref[...]
    xr_ref[...] = jnp.dot(x, wr_ref[...], preferred_element_type=jnp.float32) + br_ref[...]


def _edge_lin_body(ef_ref, we_ref, e_ref):
    e_ref[...] = jnp.dot(ef_ref[...], we_ref[...], preferred_element_type=jnp.float32)


def _epilogue_body(u_ref, d_ref, b_ref, o_ref):
    u = u_ref[0] + u_ref[1]
    d = d_ref[0] + d_ref[1]
    o_ref[...] = u / (d + 1e-16) + b_ref[...]


def _lane_sum(v):
    for sh in (8, 4, 2, 1):
        idx = lax.iota(jnp.int32, 16) ^ sh
        v = v + v.at[idx].get(mode="promise_in_bounds")
    return v


def _sc_edge_body(xl_hbm, xr_hbm, e_hbm, src_hbm, dst_hbm, att_hbm,
                  u_hbm, dn_hbm,
                  u_acc, dn_acc,
                  att_v, src_v, dst_v, dstrow_v, xl_v, xr_v, e_v,
                  sbuf_v, drow_v,
                  sem1, sem2):
    core = lax.axis_index("c")
    sub = lax.axis_index("s")
    wid = sub * NUM_CORES + core

    def zero_row(i, _):
        for j in range(D // 16):
            xl_v[i, pl.ds(j * 16, 16)] = jnp.zeros((16,), jnp.float32)
            drow_v[i, pl.ds(j * 16, 16)] = jnp.zeros((16,), jnp.float32)
        return 0

    lax.fori_loop(0, CHUNK, zero_row, 0)

    row0 = sub * ROWS_PER_TILE
    dnrow0 = sub * DN_PER_TILE
    for k in range(ROWS_PER_TILE // CHUNK):
        pltpu.sync_copy(xl_v, u_acc.at[pl.ds(row0 + k * CHUNK, CHUNK)])
    for k in range(DN_PER_TILE // CHUNK):
        pltpu.sync_copy(drow_v, dn_acc.at[pl.ds(dnrow0 + k * CHUNK, CHUNK)])
    plsc.subcore_barrier()

    pltpu.sync_copy(att_hbm, att_v)

    def do_chunk(c, _):
        base = wid * EDGES_PER_W + c * CHUNK
        pltpu.sync_copy(src_hbm.at[pl.ds(base, CHUNK)], src_v)
        pltpu.sync_copy(dst_hbm.at[pl.ds(base, CHUNK)], dst_v)
        g1 = pltpu.async_copy(xl_hbm.at[src_v], xl_v, sem1)
        g2 = pltpu.async_copy(xr_hbm.at[dst_v], xr_v, sem2)
        pltpu.sync_copy(e_hbm.at[pl.ds(base, CHUNK)], e_v)
        g1.wait()
        g2.wait()

        starts0 = list(range(0, CHUNK - 15, 16))
        if CHUNK % 16:
            starts0.append(CHUNK - 16)
        for k0 in starts0:
            sl = pl.ds(k0, 16)
            dstrow_v[sl] = lax.shift_right_logical(dst_v[sl], 3)

        def edge_row(i, _):
            acc = jnp.zeros((16,), jnp.float32)
            for j in range(D // 16):
                sl = pl.ds(j * 16, 16)
                z = xl_v[i, sl] + xr_v[i, sl] + e_v[i, sl]
                z = jnp.maximum(z, 0.2 * z)
                acc = acc + z * att_v[sl]
            s_row = jnp.exp(_lane_sum(acc))
            sbuf_v[i, pl.ds(0, 16)] = s_row
            for j in range(D // 16):
                sl = pl.ds(j * 16, 16)
                xl_v[i, sl] = s_row * xl_v[i, sl]
            return 0

        lax.fori_loop(0, CHUNK, edge_row, 0)

        lane_iota = lax.iota(jnp.int32, 16)
        for k0 in starts0:
            lanes16 = dst_v[pl.ds(k0, 16)] & 7
            for j in range(16):
                i = k0 + j
                mask = lane_iota == lanes16[j]
                srow = sbuf_v[i, pl.ds(0, 16)]
                drow_v[i, pl.ds(0, 16)] = jnp.where(mask, srow, 0.0)

        pltpu.sync_copy(xl_v, u_acc.at[dst_v], add=True)
        pltpu.sync_copy(drow_v, dn_acc.at[dstrow_v], add=True)
        return 0

    lax.fori_loop(0, NCHUNK, do_chunk, 0)
    plsc.subcore_barrier()

    pltpu.sync_copy(u_acc.at[pl.ds(row0, ROWS_PER_TILE)],
                    u_hbm.at[core, pl.ds(row0, ROWS_PER_TILE)])
    pltpu.sync_copy(dn_acc.at[pl.ds(dnrow0, DN_PER_TILE)],
                    dn_hbm.at[core, pl.ds(dnrow0, DN_PER_TILE)])


def _sc_edge_pass(xl, xr, e, src_idx, dst_idx, att_vec):
    mesh = plsc.VectorSubcoreMesh(core_axis_name="c", subcore_axis_name="s")
    kern = pl.kernel(
        _sc_edge_body,
        out_type=[
            jax.ShapeDtypeStruct((NUM_CORES, NPAD, D), jnp.float32),
            jax.ShapeDtypeStruct((NUM_CORES, DN_ROWS, D), jnp.float32),
        ],
        mesh=mesh,
        scratch_types=[
            pltpu.VMEM_SHARED((NPAD, D), jnp.float32),
            pltpu.VMEM_SHARED((DN_ROWS, D), jnp.float32),
            pltpu.VMEM((D,), jnp.float32),
            pltpu.VMEM((CHUNK,), jnp.int32),
            pltpu.VMEM((CHUNK,), jnp.int32),
            pltpu.VMEM((CHUNK,), jnp.int32),
            pltpu.VMEM((CHUNK, D), jnp.float32),
            pltpu.VMEM((CHUNK, D), jnp.float32),
            pltpu.VMEM((CHUNK, D), jnp.float32),
            pltpu.VMEM((CHUNK, 16), jnp.float32),
            pltpu.VMEM((CHUNK, D), jnp.float32),
            pltpu.SemaphoreType.DMA,
            pltpu.SemaphoreType.DMA,
        ],
    )
    return kern(xl, xr, e, src_idx, dst_idx, att_vec)


def kernel(node_feats, edge_index, edge_feats, W_l, b_l, W_r, b_r, W_e, att, bias):
    mm = pl.pallas_call(
        _lin_body,
        grid=(10,),
        in_specs=[
            pl.BlockSpec((N // 10, D), lambda i: (i, 0)),
            pl.BlockSpec((D, C), lambda i: (0, 0)),
            pl.BlockSpec((1, C), lambda i: (0, 0)),
            pl.BlockSpec((D, C), lambda i: (0, 0)),
            pl.BlockSpec((1, C), lambda i: (0, 0)),
        ],
        out_specs=[
            pl.BlockSpec((N // 10, C), lambda i: (i, 0)),
            pl.BlockSpec((N // 10, C), lambda i: (i, 0)),
        ],
        out_shape=[
            jax.ShapeDtypeStruct((N, C), jnp.float32),
            jax.ShapeDtypeStruct((N, C), jnp.float32),
        ],
    )
    xl, xr = mm(node_feats, W_l, b_l.reshape(1, C), W_r, b_r.reshape(1, C))

    e = pl.pallas_call(
        _edge_lin_body,
        grid=(40,),
        in_specs=[
            pl.BlockSpec((E // 40, ED), lambda i: (i, 0)),
            pl.BlockSpec((ED, C), lambda i: (0, 0)),
        ],
        out_specs=pl.BlockSpec((E // 40, C), lambda i: (i, 0)),
        out_shape=jax.ShapeDtypeStruct((E, C), jnp.float32),
    )(edge_feats, W_e)

    u, dn = _sc_edge_pass(xl, xr, e, edge_index[0], edge_index[1], att.reshape(C))

    d = dn[:, :, :8].reshape(NUM_CORES, NPAD)[:, :N, None]

    out = pl.pallas_call(
        _epilogue_body,
        grid=(10,),
        in_specs=[
            pl.BlockSpec((NUM_CORES, N // 10, C), lambda i: (0, i, 0)),
            pl.BlockSpec((NUM_CORES, N // 10, 1), lambda i: (0, i, 0)),
            pl.BlockSpec((1, C), lambda i: (0, 0)),
        ],
        out_specs=pl.BlockSpec((N // 10, C), lambda i: (i, 0)),
        out_shape=jax.ShapeDtypeStruct((N, C), jnp.float32),
    )(u, d, bias.reshape(1, C))
    return out


# speedup vs baseline: 6.8524x; 1.1973x over previous
import functools

import jax
import jax.numpy as jnp
from jax import lax
from jax.experimental import pallas as pl
from jax.experimental.pallas import tpu as pltpu
from jax.experimental.pallas import tpu_sc as plsc

N = 10000
E = 320000
D = 128
C = 128
ED = 16

NUM_CORES = 2
NUM_SUBCORES = 16
NW = NUM_CORES * NUM_SUBCORES
EDGES_PER_W = E // NW
CHUNK = 40
NCHUNK = EDGES_PER_W // CHUNK
IBATCH = 10
NPAD = 10240
ROWS_PER_TILE = NPAD // NUM_SUBCORES
ZROWS = 128
DN_ROWS = NPAD // 8
DN_PER_TILE = DN_ROWS // NUM_SUBCORES


def _lin_body(x_ref, wl_ref, bl_ref, wr_ref, br_ref, xl_ref, xr_ref):
    x = x_ref[...]
    xl_ref[...] = jnp.dot(x, wl_ref[...], preferred_element_type=jnp.float32) + bl_ref[...]
    xr_ref[...] = jnp.dot(x, wr_ref[...], preferred_element_type=jnp.float32) + br_ref[...]


def _edge_lin_body(ef_ref, we_ref, e_ref):
    e_ref[...] = jnp.dot(ef_ref[...], we_ref[...], preferred_element_type=jnp.float32)


def _epilogue_body(u_ref, d_ref, b_ref, o_ref):
    u = u_ref[0] + u_ref[1]
    d = d_ref[0] + d_ref[1]
    o_ref[...] = u / (d + 1e-16) + b_ref[...]


def _lane_sum(v):
    for sh in (8, 4, 2, 1):
        idx = lax.iota(jnp.int32, 16) ^ sh
        v = v + v.at[idx].get(mode="promise_in_bounds")
    return v


def _sc_edge_body(xl_hbm, xr_hbm, e_hbm, src_hbm, dst_hbm, att_hbm,
                  u_hbm, dn_hbm,
                  u_acc, dn_acc,
                  att_v, src_b, dst_b, dst_v, dstrow_v, xl_v, xr_v, e_v,
                  sbuf_v, drow_v,
                  sem1, sem2, sem3, sem4):
    core = lax.axis_index("c")
    sub = lax.axis_index("s")
    wid = sub * NUM_CORES + core

    def zero_row(i, _):
        for j in range(D // 16):
            xl_v[i, pl.ds(j * 16, 16)] = jnp.zeros((16,), jnp.float32)
            drow_v[i, pl.ds(j * 16, 16)] = jnp.zeros((16,), jnp.float32)
        return 0

    lax.fori_loop(0, CHUNK, zero_row, 0)

    row0 = sub * ROWS_PER_TILE
    dnrow0 = sub * DN_PER_TILE
    for k in range(ROWS_PER_TILE // CHUNK):
        pltpu.sync_copy(xl_v, u_acc.at[pl.ds(row0 + k * CHUNK, CHUNK)])
    for k in range(DN_PER_TILE // CHUNK):
        pltpu.sync_copy(drow_v, dn_acc.at[pl.ds(dnrow0 + k * CHUNK, CHUNK)])
    plsc.subcore_barrier()

    pltpu.sync_copy(att_hbm, att_v)

    starts0 = list(range(0, CHUNK - 15, 16))
    if CHUNK % 16:
        starts0.append(CHUNK - 16)

    def do_chunk(c, _):
        k = c % IBATCH
        off = k * CHUNK

        @pl.when(c > 0)
        def _():
            pltpu.make_async_copy(xl_v, u_acc.at[dst_v], sem3).wait()
            pltpu.make_async_copy(drow_v, dn_acc.at[dstrow_v], sem4).wait()

        for k0 in starts0:
            sl = pl.ds(k0, 16)
            d16 = dst_b[pl.ds(off + k0, 16)]
            dst_v[sl] = d16
            dstrow_v[sl] = lax.shift_right_logical(d16, 3)

        g1 = pltpu.async_copy(xl_hbm.at[src_b.at[pl.ds(off, CHUNK)]], xl_v, sem1)
        g2 = pltpu.async_copy(xr_hbm.at[dst_v], xr_v, sem2)
        base = wid * EDGES_PER_W + c * CHUNK
        pltpu.sync_copy(e_hbm.at[pl.ds(base, CHUNK)], e_v)
        g1.wait()
        g2.wait()

        def edge_row(i, _):
            acc = jnp.zeros((16,), jnp.float32)
            for j in range(D // 16):
                sl = pl.ds(j * 16, 16)
                z = xl_v[i, sl] + xr_v[i, sl] + e_v[i, sl]
                z = jnp.maximum(z, 0.2 * z)
                acc = acc + z * att_v[sl]
            s_row = jnp.exp(_lane_sum(acc))
            sbuf_v[i, pl.ds(0, 16)] = s_row
            for j in range(D // 16):
                sl = pl.ds(j * 16, 16)
                xl_v[i, sl] = s_row * xl_v[i, sl]
            return 0

        lax.fori_loop(0, CHUNK, edge_row, 0)

        lane_iota = lax.iota(jnp.int32, 16)
        for k0 in starts0:
            lanes16 = dst_v[pl.ds(k0, 16)] & 7
            for j in range(16):
                i = k0 + j
                mask = lane_iota == lanes16[j]
                srow = sbuf_v[i, pl.ds(0, 16)]
                drow_v[i, pl.ds(0, 16)] = jnp.where(mask, srow, 0.0)

        pltpu.async_copy(xl_v, u_acc.at[dst_v], sem3, add=True)
        pltpu.async_copy(drow_v, dn_acc.at[dstrow_v], sem4, add=True)
        return 0

    def do_batch(b, _):
        bbase = wid * EDGES_PER_W + b * IBATCH * CHUNK
        pltpu.sync_copy(src_hbm.at[pl.ds(bbase, IBATCH * CHUNK)], src_b)
        pltpu.sync_copy(dst_hbm.at[pl.ds(bbase, IBATCH * CHUNK)], dst_b)
        lax.fori_loop(b * IBATCH, (b + 1) * IBATCH, do_chunk, 0)
        return 0

    lax.fori_loop(0, NCHUNK // IBATCH, do_batch, 0)
    pltpu.make_async_copy(xl_v, u_acc.at[dst_v], sem3).wait()
    pltpu.make_async_copy(drow_v, dn_acc.at[dstrow_v], sem4).wait()
    plsc.subcore_barrier()

    pltpu.sync_copy(u_acc.at[pl.ds(row0, ROWS_PER_TILE)],
                    u_hbm.at[core, pl.ds(row0, ROWS_PER_TILE)])
    pltpu.sync_copy(dn_acc.at[pl.ds(dnrow0, DN_PER_TILE)],
                    dn_hbm.at[core, pl.ds(dnrow0, DN_PER_TILE)])


def _sc_edge_pass(xl, xr, e, src_idx, dst_idx, att_vec):
    mesh = plsc.VectorSubcoreMesh(core_axis_name="c", subcore_axis_name="s")
    kern = pl.kernel(
        _sc_edge_body,
        out_type=[
            jax.ShapeDtypeStruct((NUM_CORES, NPAD, D), jnp.float32),
            jax.ShapeDtypeStruct((NUM_CORES, DN_ROWS, D), jnp.float32),
        ],
        mesh=mesh,
        scratch_types=[
            pltpu.VMEM_SHARED((NPAD, D), jnp.float32),
            pltpu.VMEM_SHARED((DN_ROWS, D), jnp.float32),
            pltpu.VMEM((D,), jnp.float32),
            pltpu.VMEM((IBATCH * CHUNK,), jnp.int32),
            pltpu.VMEM((IBATCH * CHUNK,), jnp.int32),
            pltpu.VMEM((CHUNK,), jnp.int32),
            pltpu.VMEM((CHUNK,), jnp.int32),
            pltpu.VMEM((CHUNK, D), jnp.float32),
            pltpu.VMEM((CHUNK, D), jnp.float32),
            pltpu.VMEM((CHUNK, D), jnp.float32),
            pltpu.VMEM((CHUNK, 16), jnp.float32),
            pltpu.VMEM((CHUNK, D), jnp.float32),
            pltpu.SemaphoreType.DMA,
            pltpu.SemaphoreType.DMA,
            pltpu.SemaphoreType.DMA,
            pltpu.SemaphoreType.DMA,
        ],
    )
    return kern(xl, xr, e, src_idx, dst_idx, att_vec)


def kernel(node_feats, edge_index, edge_feats, W_l, b_l, W_r, b_r, W_e, att, bias):
    mm = pl.pallas_call(
        _lin_body,
        grid=(10,),
        in_specs=[
            pl.BlockSpec((N // 10, D), lambda i: (i, 0)),
            pl.BlockSpec((D, C), lambda i: (0, 0)),
            pl.BlockSpec((1, C), lambda i: (0, 0)),
            pl.BlockSpec((D, C), lambda i: (0, 0)),
            pl.BlockSpec((1, C), lambda i: (0, 0)),
        ],
        out_specs=[
            pl.BlockSpec((N // 10, C), lambda i: (i, 0)),
            pl.BlockSpec((N // 10, C), lambda i: (i, 0)),
        ],
        out_shape=[
            jax.ShapeDtypeStruct((N, C), jnp.float32),
            jax.ShapeDtypeStruct((N, C), jnp.float32),
        ],
    )
    xl, xr = mm(node_feats, W_l, b_l.reshape(1, C), W_r, b_r.reshape(1, C))

    e = pl.pallas_call(
        _edge_lin_body,
        grid=(40,),
        in_specs=[
            pl.BlockSpec((E // 40, ED), lambda i: (i, 0)),
            pl.BlockSpec((ED, C), lambda i: (0, 0)),
        ],
        out_specs=pl.BlockSpec((E // 40, C), lambda i: (i, 0)),
        out_shape=jax.ShapeDtypeStruct((E, C), jnp.float32),
    )(edge_feats, W_e)

    u, dn = _sc_edge_pass(xl, xr, e, edge_index[0], edge_index[1], att.reshape(C))

    d = dn[:, :, :8].reshape(NUM_CORES, NPAD)[:, :N, None]

    out = pl.pallas_call(
        _epilogue_body,
        grid=(10,),
        in_specs=[
            pl.BlockSpec((NUM_CORES, N // 10, C), lambda i: (0, i, 0)),
            pl.BlockSpec((NUM_CORES, N // 10, 1), lambda i: (0, i, 0)),
            pl.BlockSpec((1, C), lambda i: (0, 0)),
        ],
        out_specs=pl.BlockSpec((N // 10, C), lambda i: (i, 0)),
        out_shape=jax.ShapeDtypeStruct((N, C), jnp.float32),
    )(u, d, bias.reshape(1, C))
    return out
